# baseline (speedup 1.0000x reference)
import functools

import jax
import jax.numpy as jnp
from jax import lax
from jax.experimental import pallas as pl
from jax.experimental.pallas import tpu as pltpu
from jax.experimental.pallas import tpu_sc as plsc

_L = 16


def _sc_embed(x_flat, table, D):
    (P,) = x_flat.shape
    info = plsc.get_sparse_core_info()
    NC, NS = info.num_cores, info.num_subcores
    NW = NC * NS
    per_w = P // NW
    K = 256
    KC = 176
    KG = K - KC
    NB = 2
    KNB = K * NB
    n_outer = per_w // KNB
    n_sub = D // _L
    nqg = KG // _L

    mesh = plsc.VectorSubcoreMesh(core_axis_name="c", subcore_axis_name="s")

    @functools.partial(
        pl.kernel,
        mesh=mesh,
        out_type=jax.ShapeDtypeStruct((P, D), jnp.float32),
        scratch_types=[
            pltpu.VMEM((per_w,), jnp.float32),
            pltpu.VMEM((2, D), jnp.float32),
            pltpu.VMEM((NB, K, D), jnp.float32),
            pltpu.VMEM((NB, KG), jnp.int32),
            pltpu.VMEM_SHARED((2, D), jnp.float32),
            pltpu.SemaphoreType.DMA,
            pltpu.SemaphoreType.DMA,
            pltpu.SemaphoreType.DMA,
            pltpu.SemaphoreType.DMA,
        ],
    )
    def body(x_hbm, tbl_hbm, out_hbm, xv, tblv, rowsc, idxg, stbl,
             semo0, semo1, semg0, semg1):
        wid = lax.axis_index("s") * NC + lax.axis_index("c")
        base = wid * per_w
        pltpu.sync_copy(x_hbm.at[pl.ds(base, per_w)], xv)
        pltpu.sync_copy(tbl_hbm.at[pl.ds(0, 2)], tblv)
        @pl.when(lax.axis_index("s") == 0)
        def _stage():
            pltpu.sync_copy(tbl_hbm.at[pl.ds(0, 2)], stbl)

        plsc.subcore_barrier()
        semo = (semo0, semo1)
        semg = (semg0, semg1)

        t0 = [tblv[0, pl.ds(k * _L, _L)] for k in range(n_sub)]
        td = [tblv[1, pl.ds(k * _L, _L)] - t0[k] for k in range(n_sub)]
        ones = jnp.full((_L,), 1.0, jnp.float32)
        zeros = jnp.full((_L,), 0.0, jnp.float32)
        onei = jnp.full((_L,), 1, jnp.int32)
        zeroi = jnp.full((_L,), 0, jnp.int32)

        def chunk(c, b, wait_out):
            rows_b = rowsc.at[b]
            pos0 = c * KNB + b * K

            @pl.when(c > 0)
            def _wait():
                pltpu.make_async_copy(
                    rowsc.at[b], out_hbm.at[pl.ds(0, K)], semo[b]
                ).wait()

            for q in range(nqg):
                xq = xv[pl.ds(pos0 + KC + q * _L, _L)]
                idxg.at[b][pl.ds(q * _L, _L)] = jnp.where(
                    xq > 0.0, onei, zeroi
                )
            pltpu.async_copy(
                stbl.at[idxg.at[b]], rows_b.at[pl.ds(KC, KG)], semg[b]
            )

            def pos16(ii):
                xvec = xv[pl.ds(pos0 + ii * _L, _L)]
                svec = jnp.where(xvec > 0.0, ones, zeros)
                for j in range(_L):
                    sj = jnp.broadcast_to(svec[j], (_L,))
                    o = ii * _L + j
                    for k in range(n_sub):
                        rows_b[o, pl.ds(k * _L, _L)] = t0[k] + sj * td[k]

            plsc.parallel_loop(0, KC // _L, 1, unroll=4)(pos16)

            pltpu.make_async_copy(
                stbl.at[idxg.at[b]], rows_b.at[pl.ds(KC, KG)], semg[b]
            ).wait()
            pltpu.async_copy(
                rows_b, out_hbm.at[pl.ds(base + pos0, K)], semo[b]
            )

        def outer(c, carry):
            for b in range(NB):
                chunk(c, b, wait_out=None)
            return carry

        lax.fori_loop(0, n_outer, outer, 0)
        for b in range(NB):
            pltpu.make_async_copy(
                rowsc.at[b], out_hbm.at[pl.ds(0, K)], semo[b]
            ).wait()

    return body(x_flat, table)


def kernel(x, embedding):
    B, N = x.shape
    V, D = embedding.shape
    out = _sc_embed(x.reshape(B * N), embedding, D)
    return out.reshape(B, N, D)


# speedup vs baseline: 1.0096x; 1.0096x over previous
import functools

import jax
import jax.numpy as jnp
from jax import lax
from jax.experimental import pallas as pl
from jax.experimental.pallas import tpu as pltpu
from jax.experimental.pallas import tpu_sc as plsc

_L = 16


def _sc_embed(x_flat, table, D):
    (P,) = x_flat.shape
    info = plsc.get_sparse_core_info()
    NC, NS = info.num_cores, info.num_subcores
    NW = NC * NS
    per_w = P // NW
    K = 256
    KC = 192
    KG = K - KC
    NB = 2
    KNB = K * NB
    n_outer = per_w // KNB
    n_sub = D // _L
    nqg = KG // _L

    mesh = plsc.VectorSubcoreMesh(core_axis_name="c", subcore_axis_name="s")

    @functools.partial(
        pl.kernel,
        mesh=mesh,
        out_type=jax.ShapeDtypeStruct((P, D), jnp.float32),
        scratch_types=[
            pltpu.VMEM((per_w,), jnp.float32),
            pltpu.VMEM((2, D), jnp.float32),
            pltpu.VMEM((NB, K, D), jnp.float32),
            pltpu.VMEM((NB, KG), jnp.int32),
            pltpu.VMEM_SHARED((2, D), jnp.float32),
            pltpu.SemaphoreType.DMA,
            pltpu.SemaphoreType.DMA,
            pltpu.SemaphoreType.DMA,
            pltpu.SemaphoreType.DMA,
        ],
    )
    def body(x_hbm, tbl_hbm, out_hbm, xv, tblv, rowsc, idxg, stbl,
             semo0, semo1, semg0, semg1):
        wid = lax.axis_index("s") * NC + lax.axis_index("c")
        base = wid * per_w
        pltpu.sync_copy(x_hbm.at[pl.ds(base, per_w)], xv)
        pltpu.sync_copy(tbl_hbm.at[pl.ds(0, 2)], tblv)
        @pl.when(lax.axis_index("s") == 0)
        def _stage():
            pltpu.sync_copy(tbl_hbm.at[pl.ds(0, 2)], stbl)

        plsc.subcore_barrier()
        semo = (semo0, semo1)
        semg = (semg0, semg1)

        t0 = [tblv[0, pl.ds(k * _L, _L)] for k in range(n_sub)]
        td = [tblv[1, pl.ds(k * _L, _L)] - t0[k] for k in range(n_sub)]
        ones = jnp.full((_L,), 1.0, jnp.float32)
        zeros = jnp.full((_L,), 0.0, jnp.float32)
        onei = jnp.full((_L,), 1, jnp.int32)
        zeroi = jnp.full((_L,), 0, jnp.int32)

        def chunk(c, b):
            rows_b = rowsc.at[b]
            pos0 = c * KNB + b * K

            @pl.when(c > 0)
            def _wait():
                pltpu.make_async_copy(
                    rowsc.at[b], out_hbm.at[pl.ds(0, K)], semo[b]
                ).wait()

            for q in range(nqg):
                xq = xv[pl.ds(pos0 + KC + q * _L, _L)]
                idxg.at[b][pl.ds(q * _L, _L)] = jnp.where(
                    xq > 0.0, onei, zeroi
                )
            pltpu.async_copy(
                stbl.at[idxg.at[b]], rows_b.at[pl.ds(KC, KG)], semg[b]
            )

            def pos16(ii):
                xvec = xv[pl.ds(pos0 + ii * _L, _L)]
                svec = jnp.where(xvec > 0.0, ones, zeros)
                for j in range(_L):
                    sj = jnp.broadcast_to(svec[j], (_L,))
                    o = ii * _L + j
                    for k in range(n_sub):
                        rows_b[o, pl.ds(k * _L, _L)] = t0[k] + sj * td[k]

            plsc.parallel_loop(0, KC // _L, 1, unroll=4)(pos16)

            pltpu.make_async_copy(
                stbl.at[idxg.at[b]], rows_b.at[pl.ds(KC, KG)], semg[b]
            ).wait()
            pltpu.async_copy(
                rows_b, out_hbm.at[pl.ds(base + pos0, K)], semo[b]
            )

        def outer(c, carry):
            for b in range(NB):
                chunk(c, b)
            return carry

        lax.fori_loop(0, n_outer, outer, 0)
        for b in range(NB):
            pltpu.make_async_copy(
                rowsc.at[b], out_hbm.at[pl.ds(0, K)], semo[b]
            ).wait()

    return body(x_flat, table)


def kernel(x, embedding):
    B, N = x.shape
    V, D = embedding.shape
    out = _sc_embed(x.reshape(B * N), embedding, D)
    return out.reshape(B, N, D)
